# Initial kernel scaffold; baseline (speedup 1.0000x reference)
import jax
import jax.numpy as jnp
from jax.experimental import pallas as pl


def kernel(coordinate, value):
    raise NotImplementedError("write your pallas kernel here")


# speedup vs baseline: 9.1278x; 9.1278x over previous
import numpy as np
import jax
import jax.numpy as jnp
from jax.experimental import pallas as pl

_K = 8
_BQ = 256
_TWO_PI = 2.0 * np.pi


def _fused_kernel(tab_ref, coordT_ref, valT_ref,
                  idx_ref, sort_ref, grad_ref, umb_ref):
    N = tab_ref.shape[1]
    BQ = idx_ref.shape[2]
    q = pl.program_id(1)
    qs = q * BQ

    qtab = tab_ref[0, pl.ds(qs, BQ), :]
    qx = qtab[:, 0:1]
    qy = qtab[:, 1:2]
    kx = coordT_ref[0, 0:1, :]
    ky = coordT_ref[0, 1:2, :]
    dx = qx - kx
    dy = qy - ky
    d = dx * dx + dy * dy

    lane = jax.lax.broadcasted_iota(jnp.int32, (BQ, N), 1)
    qi = qs + jax.lax.broadcasted_iota(jnp.int32, (BQ, 1), 0)
    d = jnp.where(lane == qi, jnp.inf, d)

    tab = tab_ref[0]
    idx_cols = [qi]
    gathered = []
    for _ in range(_K - 1):
        m = jnp.min(d, axis=1, keepdims=True)
        cand = jnp.where(d == m, lane, N)
        j = jnp.min(cand, axis=1, keepdims=True)
        mask = lane == j
        g = jax.lax.dot_general(mask.astype(jnp.float32), tab,
                                (((1,), (0,)), ((), ())),
                                precision=jax.lax.Precision.HIGHEST,
                                preferred_element_type=jnp.float32)
        gathered.append(g)
        idx_cols.append(j)
        d = jnp.where(mask, jnp.inf, d)

    idx_ref[0] = jnp.concatenate(idx_cols, axis=1).T

    G = jnp.concatenate(gathered, axis=1).T.reshape(_K - 1, 8, BQ)
    px = G[:, 0, :]
    py = G[:, 1, :]
    pv = [G[:, 2 + i, :] for i in range(3)]

    qxT = coordT_ref[0, 0:1, pl.ds(qs, BQ)]
    qyT = coordT_ref[0, 1:2, pl.ds(qs, BQ)]
    qv = [valT_ref[0, i:i + 1, :] for i in range(3)]

    nx = px - qxT
    ny = py - qyT
    mv = [pv[i] - qv[i] for i in range(3)]

    phi = jnp.arctan2(ny, nx) / _TWO_PI + 0.5

    K1 = _K - 1
    arow = jax.lax.broadcasted_iota(jnp.int32, (K1, 1), 0)
    rank = jnp.zeros((K1, BQ), jnp.int32)
    for a in range(K1):
        pa = phi[a:a + 1, :]
        cnt = (phi < pa) | ((phi == pa) & (arow < a))
        r_a = jnp.sum(cnt.astype(jnp.int32), axis=0, keepdims=True)
        rank = jnp.where(arow == a, r_a, rank)

    rrow = jax.lax.broadcasted_iota(jnp.int32, (K1, 1), 0)
    sel = [rank[a:a + 1, :] == rrow for a in range(K1)]

    def srt(x):
        out = jnp.where(sel[0], x[0:1, :], 0.0)
        for a in range(1, K1):
            out = out + jnp.where(sel[a], x[a:a + 1, :], 0.0)
        return out

    sortidx = jnp.where(sel[0], 0, 0)
    for a in range(1, K1):
        sortidx = sortidx + jnp.where(sel[a], a, 0)
    sort_ref[0] = sortidx.astype(jnp.int32)

    def roll(x):
        return jnp.concatenate([x[1:, :], x[:1, :]], axis=0)

    px_s = srt(px); py_s = srt(py)
    pv_s = [srt(v) for v in pv]
    nx_s = srt(nx); ny_s = srt(ny)
    mv_s = [srt(v) for v in mv]
    phi_s = srt(phi)

    px_r = roll(px_s); py_r = roll(py_s)
    pv_r = [roll(v) for v in pv_s]
    nx_r = roll(nx_s); ny_r = roll(ny_s)
    mv_r = [roll(v) for v in mv_s]
    phi_r = roll(phi_s)

    eur = jnp.sqrt(nx_s * nx_s + ny_s * ny_s)
    eur_r = jnp.sqrt(nx_r * nx_r + ny_r * ny_r)
    sin_angle = jnp.abs(jnp.sin((phi_r - phi_s - 0.5) * _TWO_PI))

    ones = jnp.ones((K1, BQ), jnp.float32)
    chans = [qxT * ones, qyT * ones,
             qv[0] * ones, qv[1] * ones, qv[2] * ones,
             px_s, py_s, pv_s[0], pv_s[1], pv_s[2],
             nx_s, ny_s, mv_s[0], mv_s[1], mv_s[2],
             px_r, py_r, pv_r[0], pv_r[1], pv_r[2],
             nx_r, ny_r, mv_r[0], mv_r[1], mv_r[2],
             eur, eur_r, sin_angle]
    umb_ref[0] = jnp.stack(chans, axis=1)

    det = nx_s * ny_r - ny_s * nx_r
    area = 0.5 * jnp.abs(det)
    area_norm = jnp.sum(area, axis=0, keepdims=True)
    area_norm = jnp.where(area_norm == 0.0, 10000.0, area_norm)
    w = area / area_norm

    grads = []
    for i in range(3):
        ms = mv_s[i]
        mr = mv_r[i]
        c0 = ny_s * mr - ms * ny_r
        c1 = ms * nx_r - nx_s * mr
        c2 = nx_s * ny_r - ny_s * nx_r
        nrm = jnp.sqrt(c0 * c0 + c1 * c1 + c2 * c2)
        safe = jnp.where(nrm == 0.0, 1.0, nrm)
        u0 = jnp.sum((c0 / safe) * w, axis=0, keepdims=True)
        u1 = jnp.sum((c1 / safe) * w, axis=0, keepdims=True)
        u2 = jnp.sum((c2 / safe) * w, axis=0, keepdims=True)

        def comp(num, den):
            sd = jnp.where(den == 0.0, 1.0, den)
            return jnp.where(den == 0.0, 0.0, -num / sd) / 10000.0

        grads.append(comp(u0, u2))
        grads.append(comp(u1, u2))
    grad_ref[0] = jnp.concatenate(grads, axis=0)


def kernel(coordinate, value):
    B, N, _ = coordinate.shape
    tab = jnp.concatenate(
        [coordinate, value, jnp.zeros((B, N, 3), jnp.float32)], axis=-1)
    coordT = jnp.swapaxes(coordinate, 1, 2)
    valT = jnp.swapaxes(value, 1, 2)

    grid = (B, N // _BQ)
    out_shapes = (
        jax.ShapeDtypeStruct((B, _K, N), jnp.int32),
        jax.ShapeDtypeStruct((B, _K - 1, N), jnp.int32),
        jax.ShapeDtypeStruct((B, 6, N), jnp.float32),
        jax.ShapeDtypeStruct((B, _K - 1, 28, N), jnp.float32),
    )
    in_specs = [
        pl.BlockSpec((1, N, 8), lambda b, q: (b, 0, 0)),
        pl.BlockSpec((1, 2, N), lambda b, q: (b, 0, 0)),
        pl.BlockSpec((1, 3, _BQ), lambda b, q: (b, 0, q)),
    ]
    out_specs = (
        pl.BlockSpec((1, _K, _BQ), lambda b, q: (b, 0, q)),
        pl.BlockSpec((1, _K - 1, _BQ), lambda b, q: (b, 0, q)),
        pl.BlockSpec((1, 6, _BQ), lambda b, q: (b, 0, q)),
        pl.BlockSpec((1, _K - 1, 28, _BQ), lambda b, q: (b, 0, 0, q)),
    )
    idxT, sortT, gradT, umbT = pl.pallas_call(
        _fused_kernel,
        grid=grid,
        in_specs=in_specs,
        out_specs=out_specs,
        out_shape=out_shapes,
    )(tab, coordT, valT)

    gradient = jnp.swapaxes(gradT, 1, 2)
    idx = jnp.swapaxes(idxT, 1, 2)
    umbrella = jnp.transpose(umbT, (0, 3, 1, 2))
    sort_idx = jnp.swapaxes(sortT, 1, 2)
    return gradient, idx, umbrella, sort_idx


# speedup vs baseline: 9.1790x; 1.0056x over previous
import numpy as np
import jax
import jax.numpy as jnp
from jax.experimental import pallas as pl

_K = 8
_BQ = 512
_TWO_PI = 2.0 * np.pi


def _fused_kernel(tab_ref, coordT_ref, valT_ref,
                  idx_ref, sort_ref, grad_ref, umb_ref):
    N = tab_ref.shape[1]
    BQ = idx_ref.shape[2]
    q = pl.program_id(1)
    qs = q * BQ

    qtab = tab_ref[0, pl.ds(qs, BQ), :]
    qx = qtab[:, 0:1]
    qy = qtab[:, 1:2]
    kx = coordT_ref[0, 0:1, :]
    ky = coordT_ref[0, 1:2, :]
    dx = qx - kx
    dy = qy - ky
    d = dx * dx + dy * dy

    lane = jax.lax.broadcasted_iota(jnp.int32, (BQ, N), 1)
    qi = qs + jax.lax.broadcasted_iota(jnp.int32, (BQ, 1), 0)
    d = jnp.where(lane == qi, jnp.inf, d)

    tab = tab_ref[0]
    idx_cols = [qi]
    gathered = []
    for _ in range(_K - 1):
        m = jnp.min(d, axis=1, keepdims=True)
        cand = jnp.where(d == m, lane, N)
        j = jnp.min(cand, axis=1, keepdims=True)
        mask = lane == j
        g = jax.lax.dot_general(mask.astype(jnp.float32), tab,
                                (((1,), (0,)), ((), ())),
                                precision=jax.lax.Precision.HIGHEST,
                                preferred_element_type=jnp.float32)
        gathered.append(g)
        idx_cols.append(j)
        d = jnp.where(mask, jnp.inf, d)

    idx_ref[0] = jnp.concatenate(idx_cols, axis=1).T

    G = jnp.concatenate(gathered, axis=1).T.reshape(_K - 1, 8, BQ)
    px = G[:, 0, :]
    py = G[:, 1, :]
    pv = [G[:, 2 + i, :] for i in range(3)]

    qxT = coordT_ref[0, 0:1, pl.ds(qs, BQ)]
    qyT = coordT_ref[0, 1:2, pl.ds(qs, BQ)]
    qv = [valT_ref[0, i:i + 1, :] for i in range(3)]

    nx = px - qxT
    ny = py - qyT

    phi = jnp.arctan2(ny, nx) / _TWO_PI + 0.5

    K1 = _K - 1
    arow = jax.lax.broadcasted_iota(jnp.int32, (K1, 1), 0)
    rank = jnp.zeros((K1, BQ), jnp.int32)
    for a in range(K1):
        pa = phi[a:a + 1, :]
        cnt = (phi < pa) | ((phi == pa) & (arow < a))
        r_a = jnp.sum(cnt.astype(jnp.int32), axis=0, keepdims=True)
        rank = jnp.where(arow == a, r_a, rank)

    rrow = jax.lax.broadcasted_iota(jnp.int32, (K1, 1), 0)
    sel = [rank[a:a + 1, :] == rrow for a in range(K1)]

    def srt(x):
        out = jnp.where(sel[0], x[0:1, :], 0.0)
        for a in range(1, K1):
            out = out + jnp.where(sel[a], x[a:a + 1, :], 0.0)
        return out

    sortidx = jnp.where(sel[0], 0, 0)
    for a in range(1, K1):
        sortidx = sortidx + jnp.where(sel[a], a, 0)
    sort_ref[0] = sortidx.astype(jnp.int32)

    def roll(x):
        return jnp.concatenate([x[1:, :], x[:1, :]], axis=0)

    px_s = srt(px); py_s = srt(py)
    pv_s = [srt(v) for v in pv]
    nx_s = px_s - qxT; ny_s = py_s - qyT
    mv_s = [pv_s[i] - qv[i] for i in range(3)]
    phi_s = srt(phi)

    px_r = roll(px_s); py_r = roll(py_s)
    pv_r = [roll(v) for v in pv_s]
    nx_r = roll(nx_s); ny_r = roll(ny_s)
    mv_r = [roll(v) for v in mv_s]
    phi_r = roll(phi_s)

    eur = jnp.sqrt(nx_s * nx_s + ny_s * ny_s)
    eur_r = jnp.sqrt(nx_r * nx_r + ny_r * ny_r)
    sin_angle = jnp.abs(jnp.sin((phi_r - phi_s - 0.5) * _TWO_PI))

    ones = jnp.ones((K1, BQ), jnp.float32)
    chans = [qxT * ones, qyT * ones,
             qv[0] * ones, qv[1] * ones, qv[2] * ones,
             px_s, py_s, pv_s[0], pv_s[1], pv_s[2],
             nx_s, ny_s, mv_s[0], mv_s[1], mv_s[2],
             px_r, py_r, pv_r[0], pv_r[1], pv_r[2],
             nx_r, ny_r, mv_r[0], mv_r[1], mv_r[2],
             eur, eur_r, sin_angle]
    umb_ref[0] = jnp.stack(chans, axis=1)

    det = nx_s * ny_r - ny_s * nx_r
    area = 0.5 * jnp.abs(det)
    area_norm = jnp.sum(area, axis=0, keepdims=True)
    area_norm = jnp.where(area_norm == 0.0, 10000.0, area_norm)
    w = area / area_norm

    grads = []
    for i in range(3):
        ms = mv_s[i]
        mr = mv_r[i]
        c0 = ny_s * mr - ms * ny_r
        c1 = ms * nx_r - nx_s * mr
        c2 = nx_s * ny_r - ny_s * nx_r
        nrm = jnp.sqrt(c0 * c0 + c1 * c1 + c2 * c2)
        safe = jnp.where(nrm == 0.0, 1.0, nrm)
        u0 = jnp.sum((c0 / safe) * w, axis=0, keepdims=True)
        u1 = jnp.sum((c1 / safe) * w, axis=0, keepdims=True)
        u2 = jnp.sum((c2 / safe) * w, axis=0, keepdims=True)

        def comp(num, den):
            sd = jnp.where(den == 0.0, 1.0, den)
            return jnp.where(den == 0.0, 0.0, -num / sd) / 10000.0

        grads.append(comp(u0, u2))
        grads.append(comp(u1, u2))
    grad_ref[0] = jnp.concatenate(grads, axis=0)


def kernel(coordinate, value):
    B, N, _ = coordinate.shape
    tab = jnp.concatenate(
        [coordinate, value, jnp.zeros((B, N, 3), jnp.float32)], axis=-1)
    coordT = jnp.swapaxes(coordinate, 1, 2)
    valT = jnp.swapaxes(value, 1, 2)

    grid = (B, N // _BQ)
    out_shapes = (
        jax.ShapeDtypeStruct((B, _K, N), jnp.int32),
        jax.ShapeDtypeStruct((B, _K - 1, N), jnp.int32),
        jax.ShapeDtypeStruct((B, 6, N), jnp.float32),
        jax.ShapeDtypeStruct((B, _K - 1, 28, N), jnp.float32),
    )
    in_specs = [
        pl.BlockSpec((1, N, 8), lambda b, q: (b, 0, 0)),
        pl.BlockSpec((1, 2, N), lambda b, q: (b, 0, 0)),
        pl.BlockSpec((1, 3, _BQ), lambda b, q: (b, 0, q)),
    ]
    out_specs = (
        pl.BlockSpec((1, _K, _BQ), lambda b, q: (b, 0, q)),
        pl.BlockSpec((1, _K - 1, _BQ), lambda b, q: (b, 0, q)),
        pl.BlockSpec((1, 6, _BQ), lambda b, q: (b, 0, q)),
        pl.BlockSpec((1, _K - 1, 28, _BQ), lambda b, q: (b, 0, 0, q)),
    )
    idxT, sortT, gradT, umbT = pl.pallas_call(
        _fused_kernel,
        grid=grid,
        in_specs=in_specs,
        out_specs=out_specs,
        out_shape=out_shapes,
    )(tab, coordT, valT)

    gradient = jnp.swapaxes(gradT, 1, 2)
    idx = jnp.swapaxes(idxT, 1, 2)
    umbrella = jnp.transpose(umbT, (0, 3, 1, 2))
    sort_idx = jnp.swapaxes(sortT, 1, 2)
    return gradient, idx, umbrella, sort_idx


# speedup vs baseline: 18.3986x; 2.0044x over previous
import functools
import numpy as np
import jax
import jax.numpy as jnp
from jax import lax
from jax.experimental import pallas as pl
from jax.experimental.pallas import tpu as pltpu
from jax.experimental.pallas import tpu_sc as plsc

_K = 8
_BQ = 512
_TWO_PI = 2.0 * np.pi
_NW = 32
_C = 5


def _topk_kernel(coordQ_ref, coordT_ref, idx_ref):
    N = coordT_ref.shape[2]
    BQ = idx_ref.shape[1]
    b = pl.program_id(0)
    q = pl.program_id(1)
    qs = q * BQ

    qc = coordQ_ref[0]
    qx = qc[:, 0:1]
    qy = qc[:, 1:2]
    kx = coordT_ref[0, 0:1, :]
    ky = coordT_ref[0, 1:2, :]
    dx = qx - kx
    dy = qy - ky
    d = dx * dx + dy * dy

    lane = jax.lax.broadcasted_iota(jnp.int32, (BQ, N), 1)
    qi = qs + jax.lax.broadcasted_iota(jnp.int32, (BQ, 1), 0)
    d = jnp.where(lane == qi, jnp.inf, d)

    idx_cols = [qi]
    for _ in range(_K - 1):
        m = jnp.min(d, axis=1, keepdims=True)
        cand = jnp.where(d == m, lane, N)
        j = jnp.min(cand, axis=1, keepdims=True)
        idx_cols.append(j)
        d = jnp.where(cand == j, jnp.inf, d)

    idx_ref[0] = jnp.concatenate(idx_cols, axis=1)


def _sc_gather_kernel(table_ref, gidx_ref, out_ref, table_sh, idx_v, rows_v, sem):
    rows = idx_v.shape[0]
    wid = lax.axis_index("s") * 2 + lax.axis_index("c")
    base = wid * rows

    @pl.when(lax.axis_index("s") == 0)
    def _():
        pltpu.sync_copy(table_ref, table_sh)
    plsc.subcore_barrier()

    pltpu.sync_copy(gidx_ref.at[pl.ds(base, rows)], idx_v)
    pltpu.async_copy(table_sh.at[idx_v], rows_v, sem).wait()
    pltpu.sync_copy(rows_v, out_ref.at[pl.ds(base, rows)])
    plsc.subcore_barrier()


def _umbrella_kernel(gath_ref, coordT_ref, valT_ref,
                     sort_ref, grad_ref, umb_ref):
    BQ = sort_ref.shape[2]
    K1 = _K - 1

    g = gath_ref[0]
    px = g[:, 0, :]
    py = g[:, 1, :]
    pv = [g[:, 2 + i, :] for i in range(3)]

    qxT = coordT_ref[0, 0:1, :]
    qyT = coordT_ref[0, 1:2, :]
    qv = [valT_ref[0, i:i + 1, :] for i in range(3)]

    nx = px - qxT
    ny = py - qyT

    phi = jnp.arctan2(ny, nx) / _TWO_PI + 0.5

    arow = jax.lax.broadcasted_iota(jnp.int32, (K1, 1), 0)
    rank = jnp.zeros((K1, BQ), jnp.int32)
    for a in range(K1):
        pa = phi[a:a + 1, :]
        cnt = (phi < pa) | ((phi == pa) & (arow < a))
        r_a = jnp.sum(cnt.astype(jnp.int32), axis=0, keepdims=True)
        rank = jnp.where(arow == a, r_a, rank)

    sel = [rank[a:a + 1, :] == arow for a in range(K1)]

    def srt(x):
        out = jnp.where(sel[0], x[0:1, :], 0.0)
        for a in range(1, K1):
            out = out + jnp.where(sel[a], x[a:a + 1, :], 0.0)
        return out

    sortidx = jnp.where(sel[0], 0, 0)
    for a in range(1, K1):
        sortidx = sortidx + jnp.where(sel[a], a, 0)
    sort_ref[0] = sortidx.astype(jnp.int32)

    def roll(x):
        return jnp.concatenate([x[1:, :], x[:1, :]], axis=0)

    px_s = srt(px); py_s = srt(py)
    pv_s = [srt(v) for v in pv]
    nx_s = px_s - qxT; ny_s = py_s - qyT
    mv_s = [pv_s[i] - qv[i] for i in range(3)]
    phi_s = srt(phi)

    px_r = roll(px_s); py_r = roll(py_s)
    pv_r = [roll(v) for v in pv_s]
    nx_r = roll(nx_s); ny_r = roll(ny_s)
    mv_r = [roll(v) for v in mv_s]
    phi_r = roll(phi_s)

    eur = jnp.sqrt(nx_s * nx_s + ny_s * ny_s)
    eur_r = jnp.sqrt(nx_r * nx_r + ny_r * ny_r)
    sin_angle = jnp.abs(jnp.sin((phi_r - phi_s - 0.5) * _TWO_PI))

    ones = jnp.ones((K1, BQ), jnp.float32)
    chans = [qxT * ones, qyT * ones,
             qv[0] * ones, qv[1] * ones, qv[2] * ones,
             px_s, py_s, pv_s[0], pv_s[1], pv_s[2],
             nx_s, ny_s, mv_s[0], mv_s[1], mv_s[2],
             px_r, py_r, pv_r[0], pv_r[1], pv_r[2],
             nx_r, ny_r, mv_r[0], mv_r[1], mv_r[2],
             eur, eur_r, sin_angle]
    umb_ref[0] = jnp.stack(chans, axis=1)

    det = nx_s * ny_r - ny_s * nx_r
    area = 0.5 * jnp.abs(det)
    area_norm = jnp.sum(area, axis=0, keepdims=True)
    area_norm = jnp.where(area_norm == 0.0, 10000.0, area_norm)
    w = area / area_norm

    grads = []
    for i in range(3):
        ms = mv_s[i]
        mr = mv_r[i]
        c0 = ny_s * mr - ms * ny_r
        c1 = ms * nx_r - nx_s * mr
        c2 = nx_s * ny_r - ny_s * nx_r
        nrm = jnp.sqrt(c0 * c0 + c1 * c1 + c2 * c2)
        safe = jnp.where(nrm == 0.0, 1.0, nrm)
        u0 = jnp.sum((c0 / safe) * w, axis=0, keepdims=True)
        u1 = jnp.sum((c1 / safe) * w, axis=0, keepdims=True)
        u2 = jnp.sum((c2 / safe) * w, axis=0, keepdims=True)

        def comp(num, den):
            sd = jnp.where(den == 0.0, 1.0, den)
            return jnp.where(den == 0.0, 0.0, -num / sd) / 10000.0

        grads.append(comp(u0, u2))
        grads.append(comp(u1, u2))
    grad_ref[0] = jnp.concatenate(grads, axis=0)


def kernel(coordinate, value):
    B, N, _ = coordinate.shape
    K1 = _K - 1
    coordT = jnp.swapaxes(coordinate, 1, 2)
    valT = jnp.swapaxes(value, 1, 2)

    idx = pl.pallas_call(
        _topk_kernel,
        grid=(B, N // _BQ),
        in_specs=[
            pl.BlockSpec((1, _BQ, 2), lambda b, q: (b, q, 0)),
            pl.BlockSpec((1, 2, N), lambda b, q: (b, 0, 0)),
        ],
        out_specs=pl.BlockSpec((1, _BQ, _K), lambda b, q: (b, q, 0)),
        out_shape=jax.ShapeDtypeStruct((B, N, _K), jnp.int32),
    )(coordinate, coordT)

    table = jnp.concatenate(
        [coordinate, value, jnp.zeros((B, N, 3), jnp.float32)],
        axis=-1).reshape(B * N * 8)
    grow = (idx[:, :, 1:]
            + (jnp.arange(B, dtype=jnp.int32) * N)[:, None, None])
    gidx_flat = (grow.reshape(B * N * K1, 1) * 8
                 + jnp.arange(8, dtype=jnp.int32)).reshape(B * N * K1 * 8)
    words_per_w = (B * N * K1 * 8) // _NW

    mesh = plsc.VectorSubcoreMesh(core_axis_name="c", subcore_axis_name="s")
    gath = pl.kernel(
        _sc_gather_kernel,
        out_type=jax.ShapeDtypeStruct((B * N * K1 * 8,), jnp.float32),
        mesh=mesh,
        scratch_types=[
            pltpu.VMEM_SHARED((B * N * 8,), jnp.float32),
            pltpu.VMEM((words_per_w,), jnp.int32),
            pltpu.VMEM((words_per_w,), jnp.float32),
            pltpu.SemaphoreType.DMA,
        ],
    )(table, gidx_flat)

    gathT = jnp.transpose(gath.reshape(B, N, K1, 8), (0, 2, 3, 1))

    sortT, gradT, umbT = pl.pallas_call(
        _umbrella_kernel,
        grid=(B, N // _BQ),
        in_specs=[
            pl.BlockSpec((1, K1, 8, _BQ), lambda b, q: (b, 0, 0, q)),
            pl.BlockSpec((1, 2, _BQ), lambda b, q: (b, 0, q)),
            pl.BlockSpec((1, 3, _BQ), lambda b, q: (b, 0, q)),
        ],
        out_specs=(
            pl.BlockSpec((1, K1, _BQ), lambda b, q: (b, 0, q)),
            pl.BlockSpec((1, 6, _BQ), lambda b, q: (b, 0, q)),
            pl.BlockSpec((1, K1, 28, _BQ), lambda b, q: (b, 0, 0, q)),
        ),
        out_shape=(
            jax.ShapeDtypeStruct((B, K1, N), jnp.int32),
            jax.ShapeDtypeStruct((B, 6, N), jnp.float32),
            jax.ShapeDtypeStruct((B, K1, 28, N), jnp.float32),
        ),
    )(gathT, coordT, valT)

    gradient = jnp.swapaxes(gradT, 1, 2)
    umbrella = jnp.transpose(umbT, (0, 3, 1, 2))
    sort_idx = jnp.swapaxes(sortT, 1, 2)
    return gradient, idx, umbrella, sort_idx


# speedup vs baseline: 20.1208x; 1.0936x over previous
import functools
import numpy as np
import jax
import jax.numpy as jnp
from jax import lax
from jax.experimental import pallas as pl
from jax.experimental.pallas import tpu as pltpu
from jax.experimental.pallas import tpu_sc as plsc

_K = 8
_BQ = 512
_TWO_PI = 2.0 * np.pi
_NW = 32
_C = 5


def _topk_kernel(coordQ_ref, coordT_ref, idx_ref, gidx_ref):
    N = coordT_ref.shape[2]
    BQ = idx_ref.shape[1]
    b = pl.program_id(0)
    q = pl.program_id(1)
    qs = q * BQ

    qc = coordQ_ref[0]
    qx = qc[:, 0:1]
    qy = qc[:, 1:2]
    kx = coordT_ref[0, 0:1, :]
    ky = coordT_ref[0, 1:2, :]
    dx = qx - kx
    dy = qy - ky
    d = dx * dx + dy * dy

    lane = jax.lax.broadcasted_iota(jnp.int32, (BQ, N), 1)
    qi = qs + jax.lax.broadcasted_iota(jnp.int32, (BQ, 1), 0)
    d = jnp.where(lane == qi, jnp.inf, d)

    idx_cols = [qi]
    for t in range(_K - 1):
        m = jnp.min(d, axis=1, keepdims=True)
        cand = jnp.where(d == m, lane, N)
        j = jnp.min(cand, axis=1, keepdims=True)
        idx_cols.append(j)
        if t < _K - 2:
            d = jnp.where(cand == j, jnp.inf, d)

    idx_ref[0] = jnp.concatenate(idx_cols, axis=1)
    goff = b * N * 8
    gcols = [j * 8 + (goff + c) for j in idx_cols[1:] for c in range(8)]
    gidx_ref[0] = jnp.concatenate(gcols, axis=1)


def _sc_gather_kernel(table_ref, gidx_ref, out_ref, table_sh, idx_v, rows_v, sem):
    rows = idx_v.shape[0]
    wid = lax.axis_index("s") * 2 + lax.axis_index("c")
    base = wid * rows

    @pl.when(lax.axis_index("s") == 0)
    def _():
        pltpu.sync_copy(table_ref, table_sh)
    plsc.subcore_barrier()

    pltpu.sync_copy(gidx_ref.at[pl.ds(base, rows)], idx_v)
    pltpu.async_copy(table_sh.at[idx_v], rows_v, sem).wait()
    pltpu.sync_copy(rows_v, out_ref.at[pl.ds(base, rows)])
    plsc.subcore_barrier()


def _umbrella_kernel(gath_ref, coordT_ref, valT_ref,
                     sort_ref, grad_ref, umb_ref):
    BQ = sort_ref.shape[2]
    K1 = _K - 1

    g = gath_ref[0]
    px = g[:, 0, :]
    py = g[:, 1, :]
    pv = [g[:, 2 + i, :] for i in range(3)]

    qxT = coordT_ref[0, 0:1, :]
    qyT = coordT_ref[0, 1:2, :]
    qv = [valT_ref[0, i:i + 1, :] for i in range(3)]

    nx = px - qxT
    ny = py - qyT

    phi = jnp.arctan2(ny, nx) / _TWO_PI + 0.5

    arow = jax.lax.broadcasted_iota(jnp.int32, (K1, 1), 0)
    rank = jnp.zeros((K1, BQ), jnp.int32)
    for a in range(K1):
        pa = phi[a:a + 1, :]
        cnt = (phi < pa) | ((phi == pa) & (arow < a))
        r_a = jnp.sum(cnt.astype(jnp.int32), axis=0, keepdims=True)
        rank = jnp.where(arow == a, r_a, rank)

    sel = [rank[a:a + 1, :] == arow for a in range(K1)]

    def srt(x):
        out = jnp.where(sel[0], x[0:1, :], 0.0)
        for a in range(1, K1):
            out = out + jnp.where(sel[a], x[a:a + 1, :], 0.0)
        return out

    sortidx = jnp.where(sel[0], 0, 0)
    for a in range(1, K1):
        sortidx = sortidx + jnp.where(sel[a], a, 0)
    sort_ref[0] = sortidx.astype(jnp.int32)

    def roll(x):
        return jnp.concatenate([x[1:, :], x[:1, :]], axis=0)

    px_s = srt(px); py_s = srt(py)
    pv_s = [srt(v) for v in pv]
    nx_s = px_s - qxT; ny_s = py_s - qyT
    mv_s = [pv_s[i] - qv[i] for i in range(3)]
    phi_s = srt(phi)

    px_r = roll(px_s); py_r = roll(py_s)
    pv_r = [roll(v) for v in pv_s]
    nx_r = roll(nx_s); ny_r = roll(ny_s)
    mv_r = [roll(v) for v in mv_s]
    phi_r = roll(phi_s)

    eur = jnp.sqrt(nx_s * nx_s + ny_s * ny_s)
    eur_r = jnp.sqrt(nx_r * nx_r + ny_r * ny_r)
    sin_angle = jnp.abs(jnp.sin((phi_r - phi_s - 0.5) * _TWO_PI))

    ones = jnp.ones((K1, BQ), jnp.float32)
    chans = [qxT * ones, qyT * ones,
             qv[0] * ones, qv[1] * ones, qv[2] * ones,
             px_s, py_s, pv_s[0], pv_s[1], pv_s[2],
             nx_s, ny_s, mv_s[0], mv_s[1], mv_s[2],
             px_r, py_r, pv_r[0], pv_r[1], pv_r[2],
             nx_r, ny_r, mv_r[0], mv_r[1], mv_r[2],
             eur, eur_r, sin_angle]
    umb_ref[0] = jnp.stack(chans, axis=1)

    det = nx_s * ny_r - ny_s * nx_r
    area = 0.5 * jnp.abs(det)
    area_norm = jnp.sum(area, axis=0, keepdims=True)
    area_norm = jnp.where(area_norm == 0.0, 10000.0, area_norm)
    w = area / area_norm

    grads = []
    for i in range(3):
        ms = mv_s[i]
        mr = mv_r[i]
        c0 = ny_s * mr - ms * ny_r
        c1 = ms * nx_r - nx_s * mr
        c2 = nx_s * ny_r - ny_s * nx_r
        nrm = jnp.sqrt(c0 * c0 + c1 * c1 + c2 * c2)
        safe = jnp.where(nrm == 0.0, 1.0, nrm)
        u0 = jnp.sum((c0 / safe) * w, axis=0, keepdims=True)
        u1 = jnp.sum((c1 / safe) * w, axis=0, keepdims=True)
        u2 = jnp.sum((c2 / safe) * w, axis=0, keepdims=True)

        def comp(num, den):
            sd = jnp.where(den == 0.0, 1.0, den)
            return jnp.where(den == 0.0, 0.0, -num / sd) / 10000.0

        grads.append(comp(u0, u2))
        grads.append(comp(u1, u2))
    grad_ref[0] = jnp.concatenate(grads, axis=0)


def kernel(coordinate, value):
    B, N, _ = coordinate.shape
    K1 = _K - 1
    coordT = jnp.swapaxes(coordinate, 1, 2)
    valT = jnp.swapaxes(value, 1, 2)

    idx, gidx = pl.pallas_call(
        _topk_kernel,
        grid=(B, N // _BQ),
        in_specs=[
            pl.BlockSpec((1, _BQ, 2), lambda b, q: (b, q, 0)),
            pl.BlockSpec((1, 2, N), lambda b, q: (b, 0, 0)),
        ],
        out_specs=(
            pl.BlockSpec((1, _BQ, _K), lambda b, q: (b, q, 0)),
            pl.BlockSpec((1, _BQ, 56), lambda b, q: (b, q, 0)),
        ),
        out_shape=(
            jax.ShapeDtypeStruct((B, N, _K), jnp.int32),
            jax.ShapeDtypeStruct((B, N, 56), jnp.int32),
        ),
    )(coordinate, coordT)

    table = jnp.concatenate(
        [coordinate, value, jnp.zeros((B, N, 3), jnp.float32)],
        axis=-1).reshape(B * N * 8)
    gidx_flat = gidx.reshape(B * N * K1 * 8)
    words_per_w = (B * N * K1 * 8) // _NW

    mesh = plsc.VectorSubcoreMesh(core_axis_name="c", subcore_axis_name="s")
    gath = pl.kernel(
        _sc_gather_kernel,
        out_type=jax.ShapeDtypeStruct((B * N * K1 * 8,), jnp.float32),
        mesh=mesh,
        scratch_types=[
            pltpu.VMEM_SHARED((B * N * 8,), jnp.float32),
            pltpu.VMEM((words_per_w,), jnp.int32),
            pltpu.VMEM((words_per_w,), jnp.float32),
            pltpu.SemaphoreType.DMA,
        ],
    )(table, gidx_flat)

    gathT = jnp.transpose(gath.reshape(B, N, K1, 8), (0, 2, 3, 1))

    sortT, gradT, umbT = pl.pallas_call(
        _umbrella_kernel,
        grid=(B, N // _BQ),
        in_specs=[
            pl.BlockSpec((1, K1, 8, _BQ), lambda b, q: (b, 0, 0, q)),
            pl.BlockSpec((1, 2, _BQ), lambda b, q: (b, 0, q)),
            pl.BlockSpec((1, 3, _BQ), lambda b, q: (b, 0, q)),
        ],
        out_specs=(
            pl.BlockSpec((1, K1, _BQ), lambda b, q: (b, 0, q)),
            pl.BlockSpec((1, 6, _BQ), lambda b, q: (b, 0, q)),
            pl.BlockSpec((1, K1, 28, _BQ), lambda b, q: (b, 0, 0, q)),
        ),
        out_shape=(
            jax.ShapeDtypeStruct((B, K1, N), jnp.int32),
            jax.ShapeDtypeStruct((B, 6, N), jnp.float32),
            jax.ShapeDtypeStruct((B, K1, 28, N), jnp.float32),
        ),
    )(gathT, coordT, valT)

    gradient = jnp.swapaxes(gradT, 1, 2)
    umbrella = jnp.transpose(umbT, (0, 3, 1, 2))
    sort_idx = jnp.swapaxes(sortT, 1, 2)
    return gradient, idx, umbrella, sort_idx


# speedup vs baseline: 20.1663x; 1.0023x over previous
import functools
import numpy as np
import jax
import jax.numpy as jnp
from jax import lax
from jax.experimental import pallas as pl
from jax.experimental.pallas import tpu as pltpu
from jax.experimental.pallas import tpu_sc as plsc

_K = 8
_BQ = 512
_BQ1 = 1024
_TWO_PI = 2.0 * np.pi
_NW = 32
_C = 5


def _topk_kernel(coordQ_ref, coordT_ref, idx_ref, gidx_ref):
    N = coordT_ref.shape[2]
    BQ = idx_ref.shape[1]
    b = pl.program_id(0)
    q = pl.program_id(1)
    qs = q * BQ

    qc = coordQ_ref[0]
    qx = qc[:, 0:1]
    qy = qc[:, 1:2]
    kx = coordT_ref[0, 0:1, :]
    ky = coordT_ref[0, 1:2, :]
    dx = qx - kx
    dy = qy - ky
    d = dx * dx + dy * dy

    lane = jax.lax.broadcasted_iota(jnp.int32, (BQ, N), 1)
    qi = qs + jax.lax.broadcasted_iota(jnp.int32, (BQ, 1), 0)
    d = jnp.where(lane == qi, jnp.inf, d)

    idx_cols = [qi]
    for t in range(_K - 1):
        m = jnp.min(d, axis=1, keepdims=True)
        cand = jnp.where(d == m, lane, N)
        j = jnp.min(cand, axis=1, keepdims=True)
        idx_cols.append(j)
        if t < _K - 2:
            d = jnp.where(cand == j, jnp.inf, d)

    idx_ref[0] = jnp.concatenate(idx_cols, axis=1)
    goff = b * N * 8
    gcols = [j * 8 + (goff + c) for j in idx_cols[1:] for c in range(8)]
    gidx_ref[0] = jnp.concatenate(gcols, axis=1)


def _sc_gather_kernel(table_ref, gidx_ref, out_ref, table_sh, idx_v, rows_v, sem):
    rows = idx_v.shape[0]
    wid = lax.axis_index("s") * 2 + lax.axis_index("c")
    base = wid * rows

    @pl.when(lax.axis_index("s") == 0)
    def _():
        pltpu.sync_copy(table_ref, table_sh)
    plsc.subcore_barrier()

    pltpu.sync_copy(gidx_ref.at[pl.ds(base, rows)], idx_v)
    pltpu.async_copy(table_sh.at[idx_v], rows_v, sem).wait()
    pltpu.sync_copy(rows_v, out_ref.at[pl.ds(base, rows)])
    plsc.subcore_barrier()


def _umbrella_kernel(gath_ref, coordT_ref, valT_ref,
                     sort_ref, grad_ref, umb_ref):
    BQ = sort_ref.shape[2]
    K1 = _K - 1

    g = gath_ref[0]
    px = g[:, 0, :]
    py = g[:, 1, :]
    pv = [g[:, 2 + i, :] for i in range(3)]

    qxT = coordT_ref[0, 0:1, :]
    qyT = coordT_ref[0, 1:2, :]
    qv = [valT_ref[0, i:i + 1, :] for i in range(3)]

    nx = px - qxT
    ny = py - qyT

    phi = jnp.arctan2(ny, nx) / _TWO_PI + 0.5

    arow = jax.lax.broadcasted_iota(jnp.int32, (K1, 1), 0)
    rank = jnp.zeros((K1, BQ), jnp.int32)
    for a in range(K1):
        pa = phi[a:a + 1, :]
        cnt = (phi < pa) | ((phi == pa) & (arow < a))
        r_a = jnp.sum(cnt.astype(jnp.int32), axis=0, keepdims=True)
        rank = jnp.where(arow == a, r_a, rank)

    sel = [rank[a:a + 1, :] == arow for a in range(K1)]

    def srt(x):
        out = jnp.where(sel[0], x[0:1, :], 0.0)
        for a in range(1, K1):
            out = out + jnp.where(sel[a], x[a:a + 1, :], 0.0)
        return out

    sortidx = jnp.where(sel[0], 0, 0)
    for a in range(1, K1):
        sortidx = sortidx + jnp.where(sel[a], a, 0)
    sort_ref[0] = sortidx.astype(jnp.int32)

    def roll(x):
        return jnp.concatenate([x[1:, :], x[:1, :]], axis=0)

    px_s = srt(px); py_s = srt(py)
    pv_s = [srt(v) for v in pv]
    nx_s = px_s - qxT; ny_s = py_s - qyT
    mv_s = [pv_s[i] - qv[i] for i in range(3)]
    phi_s = srt(phi)

    px_r = roll(px_s); py_r = roll(py_s)
    pv_r = [roll(v) for v in pv_s]
    nx_r = roll(nx_s); ny_r = roll(ny_s)
    mv_r = [roll(v) for v in mv_s]
    phi_r = roll(phi_s)

    eur = jnp.sqrt(nx_s * nx_s + ny_s * ny_s)
    eur_r = jnp.sqrt(nx_r * nx_r + ny_r * ny_r)
    sin_angle = jnp.abs(jnp.sin((phi_r - phi_s - 0.5) * _TWO_PI))

    ones = jnp.ones((K1, BQ), jnp.float32)
    chans = [qxT * ones, qyT * ones,
             qv[0] * ones, qv[1] * ones, qv[2] * ones,
             px_s, py_s, pv_s[0], pv_s[1], pv_s[2],
             nx_s, ny_s, mv_s[0], mv_s[1], mv_s[2],
             px_r, py_r, pv_r[0], pv_r[1], pv_r[2],
             nx_r, ny_r, mv_r[0], mv_r[1], mv_r[2],
             eur, eur_r, sin_angle]
    umb_ref[0] = jnp.stack(chans, axis=1)

    det = nx_s * ny_r - ny_s * nx_r
    area = 0.5 * jnp.abs(det)
    area_norm = jnp.sum(area, axis=0, keepdims=True)
    area_norm = jnp.where(area_norm == 0.0, 10000.0, area_norm)
    w = area / area_norm

    grads = []
    for i in range(3):
        ms = mv_s[i]
        mr = mv_r[i]
        c0 = ny_s * mr - ms * ny_r
        c1 = ms * nx_r - nx_s * mr
        c2 = nx_s * ny_r - ny_s * nx_r
        nrm = jnp.sqrt(c0 * c0 + c1 * c1 + c2 * c2)
        safe = jnp.where(nrm == 0.0, 1.0, nrm)
        u0 = jnp.sum((c0 / safe) * w, axis=0, keepdims=True)
        u1 = jnp.sum((c1 / safe) * w, axis=0, keepdims=True)
        u2 = jnp.sum((c2 / safe) * w, axis=0, keepdims=True)

        def comp(num, den):
            sd = jnp.where(den == 0.0, 1.0, den)
            return jnp.where(den == 0.0, 0.0, -num / sd) / 10000.0

        grads.append(comp(u0, u2))
        grads.append(comp(u1, u2))
    grad_ref[0] = jnp.concatenate(grads, axis=0)


def kernel(coordinate, value):
    B, N, _ = coordinate.shape
    K1 = _K - 1
    coordT = jnp.swapaxes(coordinate, 1, 2)
    valT = jnp.swapaxes(value, 1, 2)

    idx, gidx = pl.pallas_call(
        _topk_kernel,
        grid=(B, N // _BQ1),
        in_specs=[
            pl.BlockSpec((1, _BQ1, 2), lambda b, q: (b, q, 0)),
            pl.BlockSpec((1, 2, N), lambda b, q: (b, 0, 0)),
        ],
        out_specs=(
            pl.BlockSpec((1, _BQ1, _K), lambda b, q: (b, q, 0)),
            pl.BlockSpec((1, _BQ1, 56), lambda b, q: (b, q, 0)),
        ),
        out_shape=(
            jax.ShapeDtypeStruct((B, N, _K), jnp.int32),
            jax.ShapeDtypeStruct((B, N, 56), jnp.int32),
        ),
    )(coordinate, coordT)

    table = jnp.concatenate(
        [coordinate, value, jnp.zeros((B, N, 3), jnp.float32)],
        axis=-1).reshape(B * N * 8)
    gidx_flat = gidx.reshape(B * N * K1 * 8)
    words_per_w = (B * N * K1 * 8) // _NW

    mesh = plsc.VectorSubcoreMesh(core_axis_name="c", subcore_axis_name="s")
    gath = pl.kernel(
        _sc_gather_kernel,
        out_type=jax.ShapeDtypeStruct((B * N * K1 * 8,), jnp.float32),
        mesh=mesh,
        scratch_types=[
            pltpu.VMEM_SHARED((B * N * 8,), jnp.float32),
            pltpu.VMEM((words_per_w,), jnp.int32),
            pltpu.VMEM((words_per_w,), jnp.float32),
            pltpu.SemaphoreType.DMA,
        ],
    )(table, gidx_flat)

    gathT = jnp.transpose(gath.reshape(B, N, K1, 8), (0, 2, 3, 1))

    sortT, gradT, umbT = pl.pallas_call(
        _umbrella_kernel,
        grid=(B, N // _BQ),
        in_specs=[
            pl.BlockSpec((1, K1, 8, _BQ), lambda b, q: (b, 0, 0, q)),
            pl.BlockSpec((1, 2, _BQ), lambda b, q: (b, 0, q)),
            pl.BlockSpec((1, 3, _BQ), lambda b, q: (b, 0, q)),
        ],
        out_specs=(
            pl.BlockSpec((1, K1, _BQ), lambda b, q: (b, 0, q)),
            pl.BlockSpec((1, 6, _BQ), lambda b, q: (b, 0, q)),
            pl.BlockSpec((1, K1, 28, _BQ), lambda b, q: (b, 0, 0, q)),
        ),
        out_shape=(
            jax.ShapeDtypeStruct((B, K1, N), jnp.int32),
            jax.ShapeDtypeStruct((B, 6, N), jnp.float32),
            jax.ShapeDtypeStruct((B, K1, 28, N), jnp.float32),
        ),
    )(gathT, coordT, valT)

    gradient = jnp.swapaxes(gradT, 1, 2)
    umbrella = jnp.transpose(umbT, (0, 3, 1, 2))
    sort_idx = jnp.swapaxes(sortT, 1, 2)
    return gradient, idx, umbrella, sort_idx


# speedup vs baseline: 22.0885x; 1.0953x over previous
import functools
import numpy as np
import jax
import jax.numpy as jnp
from jax import lax
from jax.experimental import pallas as pl
from jax.experimental.pallas import tpu as pltpu
from jax.experimental.pallas import tpu_sc as plsc

_K = 8
_BQ = 512
_BQ1 = 1024
_TWO_PI = 2.0 * np.pi
_NW = 32
_C = 5


def _topk_kernel(coordQ_ref, coordT_ref, idx_ref, gidx_ref):
    N = coordT_ref.shape[2]
    BQ = idx_ref.shape[1]
    b = pl.program_id(0)
    q = pl.program_id(1)
    qs = q * BQ

    qc = coordQ_ref[0]
    qx = qc[:, 0:1]
    qy = qc[:, 1:2]
    kx = coordT_ref[0, 0:1, :]
    ky = coordT_ref[0, 1:2, :]
    dx = qx - kx
    dy = qy - ky
    d = dx * dx + dy * dy

    lane = jax.lax.broadcasted_iota(jnp.int32, (BQ, N), 1)
    qi = qs + jax.lax.broadcasted_iota(jnp.int32, (BQ, 1), 0)
    d = jnp.where(lane == qi, jnp.inf, d)

    idx_cols = [qi]
    for t in range(_K - 1):
        m = jnp.min(d, axis=1, keepdims=True)
        cand = jnp.where(d == m, lane, N)
        j = jnp.min(cand, axis=1, keepdims=True)
        idx_cols.append(j)
        if t < _K - 2:
            d = jnp.where(cand == j, jnp.inf, d)

    idx_ref[0] = jnp.concatenate(idx_cols, axis=1)
    goff = b * N * 8
    gcols = [j * 8 + (goff + c) for j in idx_cols[1:] for c in range(8)]
    gidx_ref[0] = jnp.concatenate(gcols, axis=1)


def _sc_gather_kernel(table_ref, gidx_ref, out_ref, table_sh, idx_v, rows_v, sem):
    rows = idx_v.shape[0]
    wid = lax.axis_index("s") * 2 + lax.axis_index("c")
    base = wid * rows

    @pl.when(lax.axis_index("s") == 0)
    def _():
        pltpu.sync_copy(table_ref, table_sh)
    plsc.subcore_barrier()

    pltpu.sync_copy(gidx_ref.at[pl.ds(base, rows)], idx_v)
    pltpu.async_copy(table_sh.at[idx_v], rows_v, sem).wait()
    pltpu.sync_copy(rows_v, out_ref.at[pl.ds(base, rows)])
    plsc.subcore_barrier()


def _umbrella_kernel(gath_ref, coordT_ref, valT_ref,
                     sort_ref, grad_ref, umb_ref):
    BQ = sort_ref.shape[1]
    K1 = _K - 1

    g = gath_ref[0].T.reshape(K1, 8, BQ)
    px = g[:, 0, :]
    py = g[:, 1, :]
    pv = [g[:, 2 + i, :] for i in range(3)]

    qxT = coordT_ref[0, 0:1, :]
    qyT = coordT_ref[0, 1:2, :]
    qv = [valT_ref[0, i:i + 1, :] for i in range(3)]

    nx = px - qxT
    ny = py - qyT

    phi = jnp.arctan2(ny, nx) / _TWO_PI + 0.5

    arow = jax.lax.broadcasted_iota(jnp.int32, (K1, 1), 0)
    rank = jnp.zeros((K1, BQ), jnp.int32)
    for a in range(K1):
        pa = phi[a:a + 1, :]
        cnt = (phi < pa) | ((phi == pa) & (arow < a))
        r_a = jnp.sum(cnt.astype(jnp.int32), axis=0, keepdims=True)
        rank = jnp.where(arow == a, r_a, rank)

    sel = [rank[a:a + 1, :] == arow for a in range(K1)]

    def srt(x):
        out = jnp.where(sel[0], x[0:1, :], 0.0)
        for a in range(1, K1):
            out = out + jnp.where(sel[a], x[a:a + 1, :], 0.0)
        return out

    sortidx = jnp.where(sel[0], 0, 0)
    for a in range(1, K1):
        sortidx = sortidx + jnp.where(sel[a], a, 0)
    sort_ref[0] = sortidx.astype(jnp.int32).T

    def roll(x):
        return jnp.concatenate([x[1:, :], x[:1, :]], axis=0)

    px_s = srt(px); py_s = srt(py)
    pv_s = [srt(v) for v in pv]
    nx_s = px_s - qxT; ny_s = py_s - qyT
    mv_s = [pv_s[i] - qv[i] for i in range(3)]
    phi_s = srt(phi)

    px_r = roll(px_s); py_r = roll(py_s)
    pv_r = [roll(v) for v in pv_s]
    nx_r = roll(nx_s); ny_r = roll(ny_s)
    mv_r = [roll(v) for v in mv_s]
    phi_r = roll(phi_s)

    eur = jnp.sqrt(nx_s * nx_s + ny_s * ny_s)
    eur_r = jnp.sqrt(nx_r * nx_r + ny_r * ny_r)
    sin_angle = jnp.abs(jnp.sin((phi_r - phi_s - 0.5) * _TWO_PI))

    ones = jnp.ones((K1, BQ), jnp.float32)
    chans = [qxT * ones, qyT * ones,
             qv[0] * ones, qv[1] * ones, qv[2] * ones,
             px_s, py_s, pv_s[0], pv_s[1], pv_s[2],
             nx_s, ny_s, mv_s[0], mv_s[1], mv_s[2],
             px_r, py_r, pv_r[0], pv_r[1], pv_r[2],
             nx_r, ny_r, mv_r[0], mv_r[1], mv_r[2],
             eur, eur_r, sin_angle]
    umb_ref[0] = jnp.stack(chans, axis=1).reshape(K1 * 28, BQ).T

    det = nx_s * ny_r - ny_s * nx_r
    area = 0.5 * jnp.abs(det)
    area_norm = jnp.sum(area, axis=0, keepdims=True)
    area_norm = jnp.where(area_norm == 0.0, 10000.0, area_norm)
    w = area / area_norm

    grads = []
    for i in range(3):
        ms = mv_s[i]
        mr = mv_r[i]
        c0 = ny_s * mr - ms * ny_r
        c1 = ms * nx_r - nx_s * mr
        c2 = nx_s * ny_r - ny_s * nx_r
        nrm = jnp.sqrt(c0 * c0 + c1 * c1 + c2 * c2)
        safe = jnp.where(nrm == 0.0, 1.0, nrm)
        u0 = jnp.sum((c0 / safe) * w, axis=0, keepdims=True)
        u1 = jnp.sum((c1 / safe) * w, axis=0, keepdims=True)
        u2 = jnp.sum((c2 / safe) * w, axis=0, keepdims=True)

        def comp(num, den):
            sd = jnp.where(den == 0.0, 1.0, den)
            return jnp.where(den == 0.0, 0.0, -num / sd) / 10000.0

        grads.append(comp(u0, u2))
        grads.append(comp(u1, u2))
    grad_ref[0] = jnp.concatenate(grads, axis=0).T


def kernel(coordinate, value):
    B, N, _ = coordinate.shape
    K1 = _K - 1
    coordT = jnp.swapaxes(coordinate, 1, 2)
    valT = jnp.swapaxes(value, 1, 2)

    idx, gidx = pl.pallas_call(
        _topk_kernel,
        grid=(B, N // _BQ1),
        in_specs=[
            pl.BlockSpec((1, _BQ1, 2), lambda b, q: (b, q, 0)),
            pl.BlockSpec((1, 2, N), lambda b, q: (b, 0, 0)),
        ],
        out_specs=(
            pl.BlockSpec((1, _BQ1, _K), lambda b, q: (b, q, 0)),
            pl.BlockSpec((1, _BQ1, 56), lambda b, q: (b, q, 0)),
        ),
        out_shape=(
            jax.ShapeDtypeStruct((B, N, _K), jnp.int32),
            jax.ShapeDtypeStruct((B, N, 56), jnp.int32),
        ),
    )(coordinate, coordT)

    table = jnp.concatenate(
        [coordinate, value, jnp.zeros((B, N, 3), jnp.float32)],
        axis=-1).reshape(B * N * 8)
    gidx_flat = gidx.reshape(B * N * K1 * 8)
    words_per_w = (B * N * K1 * 8) // _NW

    mesh = plsc.VectorSubcoreMesh(core_axis_name="c", subcore_axis_name="s")
    gath = pl.kernel(
        _sc_gather_kernel,
        out_type=jax.ShapeDtypeStruct((B * N * K1 * 8,), jnp.float32),
        mesh=mesh,
        scratch_types=[
            pltpu.VMEM_SHARED((B * N * 8,), jnp.float32),
            pltpu.VMEM((words_per_w,), jnp.int32),
            pltpu.VMEM((words_per_w,), jnp.float32),
            pltpu.SemaphoreType.DMA,
        ],
    )(table, gidx_flat)

    gath2 = gath.reshape(B, N, K1 * 8)

    sort_idx, gradient, umb = pl.pallas_call(
        _umbrella_kernel,
        grid=(B, N // _BQ),
        in_specs=[
            pl.BlockSpec((1, _BQ, K1 * 8), lambda b, q: (b, q, 0)),
            pl.BlockSpec((1, 2, _BQ), lambda b, q: (b, 0, q)),
            pl.BlockSpec((1, 3, _BQ), lambda b, q: (b, 0, q)),
        ],
        out_specs=(
            pl.BlockSpec((1, _BQ, K1), lambda b, q: (b, q, 0)),
            pl.BlockSpec((1, _BQ, 6), lambda b, q: (b, q, 0)),
            pl.BlockSpec((1, _BQ, K1 * 28), lambda b, q: (b, q, 0)),
        ),
        out_shape=(
            jax.ShapeDtypeStruct((B, N, K1), jnp.int32),
            jax.ShapeDtypeStruct((B, N, 6), jnp.float32),
            jax.ShapeDtypeStruct((B, N, K1 * 28), jnp.float32),
        ),
    )(gath2, coordT, valT)

    umbrella = umb.reshape(B, N, K1, 28)
    return gradient, idx, umbrella, sort_idx
